# Initial kernel scaffold; baseline (speedup 1.0000x reference)
import functools

import jax
import jax.numpy as jnp
from jax import lax
from jax.experimental import pallas as pl
from jax.experimental.pallas import tpu as pltpu
from jax.experimental.pallas import tpu_sc as plsc

D = 128
NC = 2
NS = 16
NW = NC * NS
CHUNK = 128
CNT_W = 16
BLK = 2048


def _make_agg(nchunk: int, acc_rows: int, with_count: bool):
    rows_per_sub = acc_rows // NS
    assert rows_per_sub % CHUNK == 0
    mesh = plsc.VectorSubcoreMesh(core_axis_name="c", subcore_axis_name="s")

    out_type = [jax.ShapeDtypeStruct((NC, acc_rows, D), jnp.float32)]
    if with_count:
        out_type.append(jax.ShapeDtypeStruct((NC, acc_rows, CNT_W), jnp.float32))

    scratch_types = [
        pltpu.VMEM((nchunk, CHUNK), jnp.int32),
        pltpu.VMEM((nchunk, CHUNK), jnp.int32),
        pltpu.VMEM((CHUNK, D), jnp.float32),
        pltpu.VMEM((CHUNK, CNT_W), jnp.float32),
        pltpu.VMEM_SHARED((acc_rows, D), jnp.float32),
        pltpu.VMEM_SHARED((acc_rows, CNT_W), jnp.float32),
        pltpu.SemaphoreType.DMA,
    ]

    def body(x_hbm, srci_hbm, dsti_hbm, *rest):
        if with_count:
            out_hbm, cnt_hbm = rest[0], rest[1]
            rest = rest[2:]
        else:
            out_hbm = rest[0]
            rest = rest[1:]
        srcv, dstv, buf, ones_v, acc, cntacc, sem = rest

        cid = lax.axis_index("c")
        sid = lax.axis_index("s")
        w = cid * NS + sid

        pltpu.sync_copy(srci_hbm.at[w], srcv)
        pltpu.sync_copy(dsti_hbm.at[w], dstv)

        zero16 = jnp.zeros((16,), jnp.float32)
        one16 = jnp.ones((16,), jnp.float32)

        def _zrow(i, _):
            for j in range(D // 16):
                buf[i, pl.ds(j * 16, 16)] = zero16
            ones_v[i, pl.ds(0, 16)] = one16
            return ()

        lax.fori_loop(0, CHUNK, _zrow, ())

        base = sid * rows_per_sub
        for k in range(rows_per_sub // CHUNK):
            pltpu.sync_copy(buf.at[:, pl.ds(0, CNT_W)],
                            cntacc.at[pl.ds(base + k * CHUNK, CHUNK)])
            pltpu.sync_copy(buf, acc.at[pl.ds(base + k * CHUNK, CHUNK)])

        plsc.subcore_barrier()

        def _edge(j, _):
            pltpu.async_copy(x_hbm.at[srcv.at[j]], buf, sem).wait()
            pltpu.sync_copy(buf, acc.at[dstv.at[j]], add=True)
            if with_count:
                pltpu.sync_copy(ones_v, cntacc.at[dstv.at[j]], add=True)
            return ()

        lax.fori_loop(0, nchunk, _edge, ())

        plsc.subcore_barrier()

        pltpu.sync_copy(acc.at[pl.ds(base, rows_per_sub)],
                        out_hbm.at[cid, pl.ds(base, rows_per_sub)])
        if with_count:
            pltpu.sync_copy(cntacc.at[pl.ds(base, rows_per_sub)],
                            cnt_hbm.at[cid, pl.ds(base, rows_per_sub)])

    return pl.kernel(body, out_type=out_type, mesh=mesh,
                     scratch_types=scratch_types)


def _dense_body(do_relu, s_ref, c_ref, x_ref, wl_ref, wr_ref, b_ref, o_ref):
    s = s_ref[0] + s_ref[1]
    c = c_ref[0, :, 0:1] + c_ref[1, :, 0:1]
    mean = s / jnp.maximum(c, 1.0)
    acc = jnp.dot(mean, wl_ref[...], preferred_element_type=jnp.float32)
    acc = acc + jnp.dot(x_ref[...], wr_ref[...], preferred_element_type=jnp.float32)
    acc = acc + b_ref[...]
    if do_relu:
        acc = jnp.maximum(acc, 0.0)
    o_ref[...] = acc


def _make_dense(acc_rows: int, do_relu: bool):
    grid = (acc_rows // BLK,)
    return pl.pallas_call(
        functools.partial(_dense_body, do_relu),
        grid=grid,
        in_specs=[
            pl.BlockSpec((NC, BLK, D), lambda i: (0, i, 0)),
            pl.BlockSpec((NC, BLK, CNT_W), lambda i: (0, i, 0)),
            pl.BlockSpec((BLK, D), lambda i: (i, 0)),
            pl.BlockSpec((D, D), lambda i: (0, 0)),
            pl.BlockSpec((D, D), lambda i: (0, 0)),
            pl.BlockSpec((1, D), lambda i: (0, 0)),
        ],
        out_specs=pl.BlockSpec((BLK, D), lambda i: (i, 0)),
        out_shape=jax.ShapeDtypeStruct((acc_rows, D), jnp.float32),
    )


def kernel(x, edge_index, W1l, b1, W1r, W2l, b2, W2r):
    n = x.shape[0]
    e = edge_index.shape[1]
    acc_rows = ((n + BLK) // BLK) * BLK
    nchunk = -(-e // (NW * CHUNK))
    if nchunk % 2:
        nchunk += 1
    e_pad = NW * CHUNK * nchunk

    src = edge_index[0].astype(jnp.int32)
    dst = edge_index[1].astype(jnp.int32)
    src_p = jnp.concatenate([src, jnp.zeros((e_pad - e,), jnp.int32)])
    dst_p = jnp.concatenate([dst, jnp.full((e_pad - e,), n, jnp.int32)])
    srci = src_p.reshape(NW, nchunk, CHUNK)
    dsti = dst_p.reshape(NW, nchunk, CHUNK)

    x_pad = jnp.pad(x, ((0, acc_rows - n), (0, 0)))

    agg1 = _make_agg(nchunk, acc_rows, with_count=True)
    agg2 = _make_agg(nchunk, acc_rows, with_count=False)
    dense1 = _make_dense(acc_rows, do_relu=True)
    dense2 = _make_dense(acc_rows, do_relu=False)

    s1, cnt = agg1(x_pad, srci, dsti)
    h_pad = dense1(s1, cnt, x_pad, W1l.T, W1r.T, b1.reshape(1, D))
    s2 = agg2(h_pad, srci, dsti)
    out_pad = dense2(s2, cnt, h_pad, W2l.T, W2r.T, b2.reshape(1, D))
    return out_pad[:n]


# speedup vs baseline: 3.5529x; 3.5529x over previous
import functools

import jax
import jax.numpy as jnp
from jax import lax
from jax.experimental import pallas as pl
from jax.experimental.pallas import tpu as pltpu
from jax.experimental.pallas import tpu_sc as plsc

D = 128
NC = 2
NS = 16
NW = NC * NS
CHUNK = 128
BLK = 2048


def _make_agg(nchunk: int, acc_rows: int, with_count: bool):
    rows_per_sub = acc_rows // NS
    assert rows_per_sub % CHUNK == 0
    mesh = plsc.VectorSubcoreMesh(core_axis_name="c", subcore_axis_name="s",
                                  num_cores=NC, num_subcores=NS)

    out_type = [jax.ShapeDtypeStruct((NC, acc_rows, D), jnp.float32)]
    if with_count:
        out_type.append(jax.ShapeDtypeStruct((NC, acc_rows), jnp.float32))

    scratch_types = [
        pltpu.VMEM((nchunk, CHUNK), jnp.int32),
        pltpu.VMEM((nchunk, CHUNK), jnp.int32),
        pltpu.VMEM((CHUNK, D), jnp.float32),
        pltpu.VMEM((rows_per_sub,), jnp.float32),
        pltpu.VMEM_SHARED((acc_rows, D), jnp.float32),
        pltpu.VMEM_SHARED((acc_rows,), jnp.float32),
        pltpu.SemaphoreType.DMA,
    ]

    def body(x_hbm, srci_hbm, dsti_hbm, *rest):
        if with_count:
            out_hbm, cnt_hbm = rest[0], rest[1]
            rest = rest[2:]
        else:
            out_hbm = rest[0]
            rest = rest[1:]
        srcv, dstv, buf, ones_v, acc, cntacc, sem = rest

        cid = lax.axis_index("c")
        sid = lax.axis_index("s")
        w = cid * NS + sid

        pltpu.sync_copy(srci_hbm.at[w], srcv)
        pltpu.sync_copy(dsti_hbm.at[w], dstv)

        zero16 = jnp.zeros((16,), jnp.float32)
        one16 = jnp.ones((16,), jnp.float32)

        def _zrow(i, _):
            for j in range(D // 16):
                buf[i, pl.ds(j * 16, 16)] = zero16
            return ()

        lax.fori_loop(0, CHUNK, _zrow, ())

        def _zword(i, _):
            ones_v[pl.ds(i * 16, 16)] = zero16
            return ()

        lax.fori_loop(0, rows_per_sub // 16, _zword, ())

        base = sid * rows_per_sub
        pltpu.sync_copy(ones_v, cntacc.at[pl.ds(base, rows_per_sub)])
        for k in range(rows_per_sub // CHUNK):
            pltpu.sync_copy(buf, acc.at[pl.ds(base + k * CHUNK, CHUNK)])

        def _orow(i, _):
            ones_v[pl.ds(i * 16, 16)] = one16
            return ()

        if with_count:
            lax.fori_loop(0, CHUNK // 16, _orow, ())

        plsc.subcore_barrier()

        def _edge(j, _):
            pltpu.async_copy(x_hbm.at[srcv.at[j]], buf, sem).wait()
            pltpu.sync_copy(buf, acc.at[dstv.at[j]], add=True)
            if with_count:
                pltpu.sync_copy(ones_v.at[pl.ds(0, CHUNK)],
                                cntacc.at[dstv.at[j]], add=True)
            return ()

        lax.fori_loop(0, nchunk, _edge, ())

        plsc.subcore_barrier()

        pltpu.sync_copy(acc.at[pl.ds(base, rows_per_sub)],
                        out_hbm.at[cid, pl.ds(base, rows_per_sub)])
        if with_count:
            pltpu.sync_copy(cntacc.at[pl.ds(base, rows_per_sub)],
                            cnt_hbm.at[cid, pl.ds(base, rows_per_sub)])

    return pl.kernel(body, out_type=out_type, mesh=mesh,
                     scratch_types=scratch_types)


def _dense_body(do_relu, s_ref, c_ref, x_ref, wl_ref, wr_ref, b_ref, o_ref):
    s = s_ref[0] + s_ref[1]
    c = c_ref[0] + c_ref[1]
    mean = s / jnp.maximum(c, 1.0)
    acc = jnp.dot(mean, wl_ref[...], preferred_element_type=jnp.float32)
    acc = acc + jnp.dot(x_ref[...], wr_ref[...], preferred_element_type=jnp.float32)
    acc = acc + b_ref[...]
    if do_relu:
        acc = jnp.maximum(acc, 0.0)
    o_ref[...] = acc


def _make_dense(acc_rows: int, do_relu: bool):
    grid = (acc_rows // BLK,)
    return pl.pallas_call(
        functools.partial(_dense_body, do_relu),
        grid=grid,
        in_specs=[
            pl.BlockSpec((NC, BLK, D), lambda i: (0, i, 0)),
            pl.BlockSpec((NC, BLK, 1), lambda i: (0, i, 0)),
            pl.BlockSpec((BLK, D), lambda i: (i, 0)),
            pl.BlockSpec((D, D), lambda i: (0, 0)),
            pl.BlockSpec((D, D), lambda i: (0, 0)),
            pl.BlockSpec((1, D), lambda i: (0, 0)),
        ],
        out_specs=pl.BlockSpec((BLK, D), lambda i: (i, 0)),
        out_shape=jax.ShapeDtypeStruct((acc_rows, D), jnp.float32),
    )


def kernel(x, edge_index, W1l, b1, W1r, W2l, b2, W2r):
    n = x.shape[0]
    e = edge_index.shape[1]
    acc_rows = ((n + BLK) // BLK) * BLK
    nchunk = -(-e // (NW * CHUNK))
    if nchunk % 2:
        nchunk += 1
    e_pad = NW * CHUNK * nchunk

    src = edge_index[0].astype(jnp.int32)
    dst = edge_index[1].astype(jnp.int32)
    src_p = jnp.concatenate([src, jnp.zeros((e_pad - e,), jnp.int32)])
    dst_p = jnp.concatenate([dst, jnp.full((e_pad - e,), n, jnp.int32)])
    srci = src_p.reshape(NW, nchunk, CHUNK)
    dsti = dst_p.reshape(NW, nchunk, CHUNK)

    x_pad = jnp.pad(x, ((0, acc_rows - n), (0, 0)))

    agg1 = _make_agg(nchunk, acc_rows, with_count=True)
    agg2 = _make_agg(nchunk, acc_rows, with_count=False)
    dense1 = _make_dense(acc_rows, do_relu=True)
    dense2 = _make_dense(acc_rows, do_relu=False)

    s1, cnt = agg1(x_pad, srci, dsti)
    cnt = cnt.reshape(NC, acc_rows, 1)
    h_pad = dense1(s1, cnt, x_pad, W1l.T, W1r.T, b1.reshape(1, D))
    [s2] = agg2(h_pad, srci, dsti)
    out_pad = dense2(s2, cnt, h_pad, W2l.T, W2r.T, b2.reshape(1, D))
    return out_pad[:n]
